# Initial kernel scaffold; baseline (speedup 1.0000x reference)
import functools

import jax
import jax.numpy as jnp
from jax import lax
from jax.experimental import pallas as pl
from jax.experimental.pallas import tpu as pltpu
from jax.experimental.pallas import tpu_sc as plsc

NUM_USER = 30000
NUM_GROUP = 20000
N = NUM_USER + NUM_GROUP
E = 800000
D = 64
HD = D // 2
LAYERS = 3
B = 4096

NC = 2
NS = 16
CHUNK = 128
NCHUNKS = E // CHUNK
ROWS_PER_TILE = N // NS
RCHUNK = 125
BCHUNK = 128


def _mul16(rows_ref, val_ref, n):
    def body(e, _):
        v = val_ref[e]
        rows_ref[e, pl.ds(0, 16)] = rows_ref[e, pl.ds(0, 16)] * v
        rows_ref[e, pl.ds(16, 16)] = rows_ref[e, pl.ds(16, 16)] * v
        return 0
    lax.fori_loop(0, n, body, 0)


def _add_base(dst_ref, src_ref, base):
    for i in range(8):
        s = pl.ds(i * 16, 16)
        dst_ref[s] = src_ref[s] + base


def _sc_body(all0_cs, user_table, group_table, adj_val, adj_row, adj_col,
             user_inputs, pos_groups, neg_groups,
             u_cs, p_cs, n_cs, u_ego, p_ego, n_ego, layers_hbm,
             colbuf, rowbuf, offbuf, valbuf, rows, tmpv, zerov,
             rows4, rowsum, bidx, boff, egorows, acc_sh, gsem):
    c = lax.axis_index("c")
    s = lax.axis_index("s")

    def zinit(i, _):
        zerov[i, pl.ds(0, 16)] = jnp.zeros((16,), jnp.float32)
        zerov[i, pl.ds(16, 16)] = jnp.zeros((16,), jnp.float32)
        return 0
    lax.fori_loop(0, RCHUNK, zinit, 0)

    nchunks_tile = (NCHUNKS - s + NS - 1) // NS

    for l in range(1, LAYERS + 1):
        r_lo = s * ROWS_PER_TILE
        for k in range(ROWS_PER_TILE // RCHUNK):
            pltpu.sync_copy(zerov, acc_sh.at[pl.ds(r_lo + k * RCHUNK, RCHUNK)])
        plsc.subcore_barrier()

        if l == 1:
            src_ref = all0_cs
            src_base = c * N
        else:
            src_ref = layers_hbm
            src_base = ((l - 2) * NC + c) * N

        def chunk_body(ci, _):
            e0 = (s + ci * NS) * CHUNK
            pltpu.sync_copy(adj_col.at[pl.ds(e0, CHUNK)], colbuf)
            pltpu.sync_copy(adj_row.at[pl.ds(e0, CHUNK)], rowbuf.at[0])
            pltpu.sync_copy(adj_val.at[pl.ds(e0, CHUNK)], valbuf)
            _add_base(offbuf, colbuf, src_base)
            pltpu.async_copy(src_ref.at[offbuf], rows, gsem).wait()
            _mul16(rows, valbuf, CHUNK)
            pltpu.sync_copy(rows, acc_sh.at[rowbuf.at[0]], add=True)
            return 0
        lax.fori_loop(0, nchunks_tile, chunk_body, 0)
        plsc.subcore_barrier()

        dst_base = ((l - 1) * NC + c) * N
        for k in range(ROWS_PER_TILE // RCHUNK):
            r0 = r_lo + k * RCHUNK
            pltpu.sync_copy(acc_sh.at[pl.ds(r0, RCHUNK)], tmpv)
            pltpu.sync_copy(tmpv, layers_hbm.at[pl.ds(dst_base + r0, RCHUNK)])
        plsc.subcore_barrier()

    for idx_hbm, node_base, out_cs in (
        (user_inputs, 0, u_cs),
        (pos_groups, NUM_USER, p_cs),
        (neg_groups, NUM_USER, n_cs),
    ):
        for j in range(B // BCHUNK // NS):
            b0 = (s * (B // BCHUNK // NS) + j) * BCHUNK
            pltpu.sync_copy(idx_hbm.at[pl.ds(b0, BCHUNK)], bidx)
            for src_id in range(4):
                if src_id == 0:
                    base = node_base + c * N
                    src = all0_cs
                else:
                    base = node_base + ((src_id - 1) * NC + c) * N
                    src = layers_hbm
                _add_base(boff, bidx, base)
                pltpu.async_copy(src.at[boff], rows4.at[src_id], gsem).wait()

            def comb(r, _):
                for g in range(2):
                    sl = pl.ds(g * 16, 16)
                    acc = (rows4[0, r, sl] + rows4[1, r, sl]) + \
                          (rows4[2, r, sl] + rows4[3, r, sl])
                    rowsum[r, sl] = acc * 0.25
                return 0
            lax.fori_loop(0, BCHUNK, comb, 0)
            pltpu.sync_copy(rowsum, out_cs.at[c, pl.ds(b0, BCHUNK)])

    w = s * NC + c
    b0 = w * BCHUNK
    for idx_hbm, table, out in (
        (user_inputs, user_table, u_ego),
        (pos_groups, group_table, p_ego),
        (neg_groups, group_table, n_ego),
    ):
        pltpu.sync_copy(idx_hbm.at[pl.ds(b0, BCHUNK)], bidx)
        pltpu.async_copy(table.at[bidx], egorows, gsem).wait()
        pltpu.sync_copy(egorows, out.at[pl.ds(b0, BCHUNK)])


_mesh = plsc.VectorSubcoreMesh(core_axis_name="c", subcore_axis_name="s",
                               num_cores=NC, num_subcores=NS)

_sc_call = pl.kernel(
    _sc_body,
    out_type=(
        jax.ShapeDtypeStruct((NC, B, HD), jnp.float32),
        jax.ShapeDtypeStruct((NC, B, HD), jnp.float32),
        jax.ShapeDtypeStruct((NC, B, HD), jnp.float32),
        jax.ShapeDtypeStruct((B, D), jnp.float32),
        jax.ShapeDtypeStruct((B, D), jnp.float32),
        jax.ShapeDtypeStruct((B, D), jnp.float32),
        jax.ShapeDtypeStruct((LAYERS * NC * N, HD), jnp.float32),
    ),
    mesh=_mesh,
    scratch_types=[
        pltpu.VMEM((CHUNK,), jnp.int32),
        pltpu.VMEM((1, CHUNK), jnp.int32),
        pltpu.VMEM((CHUNK,), jnp.int32),
        pltpu.VMEM((CHUNK,), jnp.float32),
        pltpu.VMEM((CHUNK, HD), jnp.float32),
        pltpu.VMEM((RCHUNK, HD), jnp.float32),
        pltpu.VMEM((RCHUNK, HD), jnp.float32),
        pltpu.VMEM((4, BCHUNK, HD), jnp.float32),
        pltpu.VMEM((BCHUNK, HD), jnp.float32),
        pltpu.VMEM((BCHUNK,), jnp.int32),
        pltpu.VMEM((BCHUNK,), jnp.int32),
        pltpu.VMEM((BCHUNK, D), jnp.float32),
        pltpu.VMEM_SHARED((N, HD), jnp.float32),
        pltpu.SemaphoreType.DMA,
    ],
)


@jax.jit
def kernel(user_table, group_table, adj_val, adj_row, adj_col,
           user_inputs, pos_groups, neg_groups):
    all0 = jnp.concatenate([user_table, group_table], axis=0)
    all0_cs = all0.reshape(N, NC, HD).transpose(1, 0, 2).reshape(NC * N, HD)
    u_cs, p_cs, n_cs, u_ego, p_ego, n_ego, _ = _sc_call(
        all0_cs, user_table, group_table, adj_val, adj_row, adj_col,
        user_inputs, pos_groups, neg_groups)
    user_embeds = jnp.concatenate([u_cs[0], u_cs[1]], axis=1)
    pos_embeds = jnp.concatenate([p_cs[0], p_cs[1]], axis=1)
    neg_embeds = jnp.concatenate([n_cs[0], n_cs[1]], axis=1)
    return (user_embeds, pos_embeds, neg_embeds, u_ego, p_ego, n_ego)


# speedup vs baseline: 3.6095x; 3.6095x over previous
import functools

import jax
import jax.numpy as jnp
from jax import lax
from jax.experimental import pallas as pl
from jax.experimental.pallas import tpu as pltpu
from jax.experimental.pallas import tpu_sc as plsc

NUM_USER = 30000
NUM_GROUP = 20000
N = NUM_USER + NUM_GROUP
E = 800000
D = 64
HD = D // 2
LAYERS = 3
B = 4096

NC = 2
NS = 16
CHUNK = 128
NCHUNKS = E // CHUNK
RCHUNK = 200
NRCHUNKS = N // RCHUNK
EGO_CHUNK = 64
BCHUNK = 128


def _mul16(rows_ref, val_ref, n):
    def body(i, _):
        e0 = i * 16
        vals = val_ref[pl.ds(e0, 16)]
        for j in range(16):
            v = vals[j]
            e = e0 + j
            rows_ref[e, pl.ds(0, 16)] = rows_ref[e, pl.ds(0, 16)] * v
            rows_ref[e, pl.ds(16, 16)] = rows_ref[e, pl.ds(16, 16)] * v
        return 0
    lax.fori_loop(0, n // 16, body, 0)


def _add_base(dst_ref, src_ref, base):
    for i in range(8):
        s = pl.ds(i * 16, 16)
        dst_ref[s] = src_ref[s] + base


def _sc_body(all0_cs, user_table, group_table, adj_val, adj_row, adj_col,
             user_inputs, pos_groups, neg_groups,
             u_cs, p_cs, n_cs, u_ego, p_ego, n_ego, layers_hbm,
             colbuf, rowbuf, offbuf, valbuf, rows, tmpv,
             rowsum, bidx, boff, ebidx, egorows, acc_sh, gsem):
    c = lax.axis_index("c")
    s = lax.axis_index("s")

    nchunks_tile = (NCHUNKS - s + NS - 1) // NS
    nrchunks_tile = (NRCHUNKS - s + NS - 1) // NS

    for l in range(1, LAYERS + 1):
        def zinit(i, _):
            tmpv[i, pl.ds(0, 16)] = jnp.zeros((16,), jnp.float32)
            tmpv[i, pl.ds(16, 16)] = jnp.zeros((16,), jnp.float32)
            return 0
        lax.fori_loop(0, RCHUNK, zinit, 0)

        def zbody(k, _):
            r0 = (s + k * NS) * RCHUNK
            pltpu.sync_copy(tmpv, acc_sh.at[pl.ds(r0, RCHUNK)])
            return 0
        lax.fori_loop(0, nrchunks_tile, zbody, 0)
        plsc.subcore_barrier()

        if l == 1:
            src_ref = all0_cs
            src_base = c * N
        else:
            src_ref = layers_hbm
            src_base = ((l - 2) * NC + c) * N

        def chunk_body(ci, _):
            e0 = (s + ci * NS) * CHUNK
            pltpu.sync_copy(adj_col.at[pl.ds(e0, CHUNK)], colbuf)
            pltpu.sync_copy(adj_row.at[pl.ds(e0, CHUNK)], rowbuf.at[0])
            pltpu.sync_copy(adj_val.at[pl.ds(e0, CHUNK)], valbuf)
            _add_base(offbuf, colbuf, src_base)
            pltpu.async_copy(src_ref.at[offbuf], rows, gsem).wait()
            _mul16(rows, valbuf, CHUNK)
            pltpu.sync_copy(rows, acc_sh.at[rowbuf.at[0]], add=True)
            return 0
        lax.fori_loop(0, nchunks_tile, chunk_body, 0)
        plsc.subcore_barrier()

        dst_base = ((l - 1) * NC + c) * N
        def fbody(k, _):
            r0 = (s + k * NS) * RCHUNK
            pltpu.sync_copy(acc_sh.at[pl.ds(r0, RCHUNK)], tmpv)
            pltpu.sync_copy(tmpv, layers_hbm.at[pl.ds(dst_base + r0, RCHUNK)])
            return 0
        lax.fori_loop(0, nrchunks_tile, fbody, 0)
        plsc.subcore_barrier()

    for idx_hbm, node_base, out_cs in (
        (user_inputs, 0, u_cs),
        (pos_groups, NUM_USER, p_cs),
        (neg_groups, NUM_USER, n_cs),
    ):
        for j in range(B // BCHUNK // NS):
            b0 = (s * (B // BCHUNK // NS) + j) * BCHUNK
            pltpu.sync_copy(idx_hbm.at[pl.ds(b0, BCHUNK)], bidx)
            for src_id in range(4):
                if src_id == 0:
                    base = node_base + c * N
                    src = all0_cs
                else:
                    base = node_base + ((src_id - 1) * NC + c) * N
                    src = layers_hbm
                _add_base(boff, bidx, base)
                pltpu.async_copy(src.at[boff], rows, gsem).wait()

                def comb(r, _):
                    for g in range(2):
                        sl = pl.ds(g * 16, 16)
                        if src_id == 0:
                            rowsum[r, sl] = rows[r, sl]
                        elif src_id == 3:
                            rowsum[r, sl] = (rowsum[r, sl] + rows[r, sl]) * 0.25
                        else:
                            rowsum[r, sl] = rowsum[r, sl] + rows[r, sl]
                    return 0
                lax.fori_loop(0, BCHUNK, comb, 0)
            pltpu.sync_copy(rowsum, out_cs.at[c, pl.ds(b0, BCHUNK)])

    w = s * NC + c
    for idx_hbm, table, out in (
        (user_inputs, user_table, u_ego),
        (pos_groups, group_table, p_ego),
        (neg_groups, group_table, n_ego),
    ):
        for h in range(BCHUNK // EGO_CHUNK):
            b0 = w * BCHUNK + h * EGO_CHUNK
            pltpu.sync_copy(idx_hbm.at[pl.ds(b0, EGO_CHUNK)], ebidx)
            pltpu.async_copy(table.at[ebidx], egorows, gsem).wait()
            pltpu.sync_copy(egorows, out.at[pl.ds(b0, EGO_CHUNK)])


_mesh = plsc.VectorSubcoreMesh(core_axis_name="c", subcore_axis_name="s",
                               num_cores=NC, num_subcores=NS)

_sc_call = pl.kernel(
    _sc_body,
    out_type=(
        jax.ShapeDtypeStruct((NC, B, HD), jnp.float32),
        jax.ShapeDtypeStruct((NC, B, HD), jnp.float32),
        jax.ShapeDtypeStruct((NC, B, HD), jnp.float32),
        jax.ShapeDtypeStruct((B, D), jnp.float32),
        jax.ShapeDtypeStruct((B, D), jnp.float32),
        jax.ShapeDtypeStruct((B, D), jnp.float32),
        jax.ShapeDtypeStruct((LAYERS * NC * N, HD), jnp.float32),
    ),
    mesh=_mesh,
    scratch_types=[
        pltpu.VMEM((CHUNK,), jnp.int32),
        pltpu.VMEM((1, CHUNK), jnp.int32),
        pltpu.VMEM((CHUNK,), jnp.int32),
        pltpu.VMEM((CHUNK,), jnp.float32),
        pltpu.VMEM((CHUNK, HD), jnp.float32),
        pltpu.VMEM((RCHUNK, HD), jnp.float32),
        pltpu.VMEM((BCHUNK, HD), jnp.float32),
        pltpu.VMEM((BCHUNK,), jnp.int32),
        pltpu.VMEM((BCHUNK,), jnp.int32),
        pltpu.VMEM((EGO_CHUNK,), jnp.int32),
        pltpu.VMEM((EGO_CHUNK, D), jnp.float32),
        pltpu.VMEM_SHARED((N, HD), jnp.float32),
        pltpu.SemaphoreType.DMA,
    ],
    compiler_params=pltpu.CompilerParams(use_tc_tiling_on_sc=False),
)


@jax.jit
def kernel(user_table, group_table, adj_val, adj_row, adj_col,
           user_inputs, pos_groups, neg_groups):
    all0 = jnp.concatenate([user_table, group_table], axis=0)
    all0_cs = all0.reshape(N, NC, HD).transpose(1, 0, 2).reshape(NC * N, HD)
    u_cs, p_cs, n_cs, u_ego, p_ego, n_ego, _ = _sc_call(
        all0_cs, user_table, group_table, adj_val, adj_row, adj_col,
        user_inputs, pos_groups, neg_groups)
    user_embeds = jnp.concatenate([u_cs[0], u_cs[1]], axis=1)
    pos_embeds = jnp.concatenate([p_cs[0], p_cs[1]], axis=1)
    neg_embeds = jnp.concatenate([n_cs[0], n_cs[1]], axis=1)
    return (user_embeds, pos_embeds, neg_embeds, u_ego, p_ego, n_ego)


# speedup vs baseline: 7.7901x; 2.1582x over previous
import jax
import jax.numpy as jnp
from jax import lax
from jax.experimental import pallas as pl
from jax.experimental.pallas import tpu as pltpu
from jax.experimental.pallas import tpu_sc as plsc

NUM_USER = 30000
NUM_GROUP = 20000
N = NUM_USER + NUM_GROUP
E = 800000
D = 64
HD = D // 2
LAYERS = 3
B = 4096

NC = 2
NS = 16
CHUNK = 128
SETS = 4
TILE_EDGES = E // NS
NCHUNKS_TILE = 392
TILE_SPAN = NCHUNKS_TILE * CHUNK
E_PAD = NS * TILE_SPAN + 2 * CHUNK
RCHUNK = 100
NRCHUNKS = N // RCHUNK
EGO_CHUNK = 64
BCHUNK = 128


def _mul16(rows_ref, val_ref, n):
    def body(i, _):
        e0 = i * 16
        vals = val_ref[pl.ds(e0, 16)]
        for j in range(16):
            v = vals[j]
            e = e0 + j
            rows_ref[e, pl.ds(0, 16)] = rows_ref[e, pl.ds(0, 16)] * v
            rows_ref[e, pl.ds(16, 16)] = rows_ref[e, pl.ds(16, 16)] * v
        return 0
    lax.fori_loop(0, n // 16, body, 0)


def _add_base(dst_ref, src_ref, base):
    for i in range(8):
        s = pl.ds(i * 16, 16)
        dst_ref[s] = src_ref[s] + base


def _sc_body(all0_cs, user_table, group_table, adj_valp, adj_rowp2, adj_colp,
             user_inputs, pos_groups, neg_groups,
             u_cs, p_cs, n_cs, u_ego, p_ego, n_ego, layers_hbm,
             col0, col1, col2, col3, row0, row1, row2, row3,
             val0, val1, val2, val3, rows0, rows1, rows2, rows3,
             tmpv, rowsum, bidx, boff, ebidx, egorows, acc_sh,
             semI0, semI1, semI2, semI3, semG0, semG1, semG2, semG3,
             semS0, semS1, semS2, semS3, gsem):
    c = lax.axis_index("c")
    s = lax.axis_index("s")

    cols = [col0, col1, col2, col3]
    rowsb = [row0, row1, row2, row3]
    vals = [val0, val1, val2, val3]
    rowsd = [rows0, rows1, rows2, rows3]
    semI = [semI0, semI1, semI2, semI3]
    semG = [semG0, semG1, semG2, semG3]
    semS = [semS0, semS1, semS2, semS3]

    nrchunks_tile = (NRCHUNKS - s + NS - 1) // NS
    base_e = s * TILE_SPAN
    base_ch = s * NCHUNKS_TILE

    def issue_idx(i, X):
        e0 = base_e + i * CHUNK
        pltpu.async_copy(adj_colp.at[pl.ds(e0, CHUNK)], cols[X], semI[X])
        pltpu.async_copy(adj_rowp2.at[pl.ds(base_ch + i, 1)], rowsb[X], semI[X])
        pltpu.async_copy(adj_valp.at[pl.ds(e0, CHUNK)], vals[X], semI[X])

    def drain_idx(X):
        pltpu.make_async_copy(adj_colp.at[pl.ds(0, CHUNK)], cols[X], semI[X]).wait()
        pltpu.make_async_copy(adj_rowp2.at[pl.ds(0, 1)], rowsb[X], semI[X]).wait()
        pltpu.make_async_copy(adj_valp.at[pl.ds(0, CHUNK)], vals[X], semI[X]).wait()

    def zero_rows(X):
        def zr(r, _):
            rowsd[X][r, pl.ds(0, 16)] = jnp.zeros((16,), jnp.float32)
            rowsd[X][r, pl.ds(16, 16)] = jnp.zeros((16,), jnp.float32)
            return 0
        lax.fori_loop(0, CHUNK, zr, 0)
        for ii in range(8):
            rowsb[X][0, pl.ds(ii * 16, 16)] = jnp.zeros((16,), jnp.int32)

    for l in range(1, LAYERS + 1):
        def zinit(i, _):
            tmpv[i, pl.ds(0, 16)] = jnp.zeros((16,), jnp.float32)
            tmpv[i, pl.ds(16, 16)] = jnp.zeros((16,), jnp.float32)
            return 0
        lax.fori_loop(0, RCHUNK, zinit, 0)

        def zbody(k, _):
            r0 = (s + k * NS) * RCHUNK
            pltpu.sync_copy(tmpv, acc_sh.at[pl.ds(r0, RCHUNK)])
            return 0
        lax.fori_loop(0, nrchunks_tile, zbody, 0)
        plsc.subcore_barrier()

        if l == 1:
            src_ref = all0_cs
            src_base = c * N
        else:
            src_ref = layers_hbm
            src_base = ((l - 2) * NC + c) * N

        def offs(X):
            for ii in range(8):
                sl = pl.ds(ii * 16, 16)
                cols[X][sl] = cols[X][sl] + src_base

        def issue_gather(X):
            pltpu.async_copy(src_ref.at[cols[X]], rowsd[X], semG[X])

        def drain_gather(X):
            pltpu.make_async_copy(src_ref.at[cols[X]], rowsd[X], semG[X]).wait()

        def issue_scatter(X):
            pltpu.async_copy(rowsd[X], acc_sh.at[rowsb[X].at[0]], semS[X],
                             add=True)

        def drain_scatter(X):
            pltpu.make_async_copy(rowsd[X], acc_sh.at[rowsb[X].at[0]],
                                  semS[X]).wait()

        for X in (2, 3):
            zero_rows(X)
            issue_scatter(X)
        issue_idx(0, 0)
        issue_idx(1, 1)
        drain_idx(0)
        offs(0)
        issue_gather(0)

        def round_body(k, _):
            for u in range(SETS):
                i = k * SETS + u
                X, Y, P = u, (u + 1) % SETS, (u + 2) % SETS
                drain_gather(X)
                _mul16(rowsd[X], vals[X], CHUNK)
                issue_scatter(X)
                drain_idx(Y)
                offs(Y)
                drain_scatter(P)
                issue_gather(Y)
                issue_idx(i + 2, P)
            return 0
        lax.fori_loop(0, NCHUNKS_TILE // SETS, round_body, 0)

        drain_scatter(2)
        drain_scatter(3)
        drain_gather(0)
        drain_idx(1)
        plsc.subcore_barrier()

        dst_base = ((l - 1) * NC + c) * N
        def fbody(k, _):
            r0 = (s + k * NS) * RCHUNK
            pltpu.sync_copy(acc_sh.at[pl.ds(r0, RCHUNK)], tmpv)
            pltpu.sync_copy(tmpv, layers_hbm.at[pl.ds(dst_base + r0, RCHUNK)])
            return 0
        lax.fori_loop(0, nrchunks_tile, fbody, 0)
        plsc.subcore_barrier()

    for idx_hbm, node_base, out_cs in (
        (user_inputs, 0, u_cs),
        (pos_groups, NUM_USER, p_cs),
        (neg_groups, NUM_USER, n_cs),
    ):
        for j in range(B // BCHUNK // NS):
            b0 = (s * (B // BCHUNK // NS) + j) * BCHUNK
            pltpu.sync_copy(idx_hbm.at[pl.ds(b0, BCHUNK)], bidx)
            for src_id in range(4):
                if src_id == 0:
                    base = node_base + c * N
                    src = all0_cs
                else:
                    base = node_base + ((src_id - 1) * NC + c) * N
                    src = layers_hbm
                _add_base(boff, bidx, base)
                pltpu.async_copy(src.at[boff], rows0, gsem).wait()

                def comb(r, _):
                    for g in range(2):
                        sl = pl.ds(g * 16, 16)
                        if src_id == 0:
                            rowsum[r, sl] = rows0[r, sl]
                        elif src_id == 3:
                            rowsum[r, sl] = (rowsum[r, sl] + rows0[r, sl]) * 0.25
                        else:
                            rowsum[r, sl] = rowsum[r, sl] + rows0[r, sl]
                    return 0
                lax.fori_loop(0, BCHUNK, comb, 0)
            pltpu.sync_copy(rowsum, out_cs.at[c, pl.ds(b0, BCHUNK)])

    w = s * NC + c
    for idx_hbm, table, out in (
        (user_inputs, user_table, u_ego),
        (pos_groups, group_table, p_ego),
        (neg_groups, group_table, n_ego),
    ):
        for h in range(BCHUNK // EGO_CHUNK):
            b0 = w * BCHUNK + h * EGO_CHUNK
            pltpu.sync_copy(idx_hbm.at[pl.ds(b0, EGO_CHUNK)], ebidx)
            pltpu.async_copy(table.at[ebidx], egorows, gsem).wait()
            pltpu.sync_copy(egorows, out.at[pl.ds(b0, EGO_CHUNK)])


_mesh = plsc.VectorSubcoreMesh(core_axis_name="c", subcore_axis_name="s",
                               num_cores=NC, num_subcores=NS)

_sc_call = pl.kernel(
    _sc_body,
    out_type=(
        jax.ShapeDtypeStruct((NC, B, HD), jnp.float32),
        jax.ShapeDtypeStruct((NC, B, HD), jnp.float32),
        jax.ShapeDtypeStruct((NC, B, HD), jnp.float32),
        jax.ShapeDtypeStruct((B, D), jnp.float32),
        jax.ShapeDtypeStruct((B, D), jnp.float32),
        jax.ShapeDtypeStruct((B, D), jnp.float32),
        jax.ShapeDtypeStruct((LAYERS * NC * N, HD), jnp.float32),
    ),
    mesh=_mesh,
    scratch_types=(
        [pltpu.VMEM((CHUNK,), jnp.int32) for _ in range(SETS)] +
        [pltpu.VMEM((1, CHUNK), jnp.int32) for _ in range(SETS)] +
        [pltpu.VMEM((CHUNK,), jnp.float32) for _ in range(SETS)] +
        [pltpu.VMEM((CHUNK, HD), jnp.float32) for _ in range(SETS)] +
        [
            pltpu.VMEM((RCHUNK, HD), jnp.float32),
            pltpu.VMEM((BCHUNK, HD), jnp.float32),
            pltpu.VMEM((BCHUNK,), jnp.int32),
            pltpu.VMEM((BCHUNK,), jnp.int32),
            pltpu.VMEM((EGO_CHUNK,), jnp.int32),
            pltpu.VMEM((EGO_CHUNK, D), jnp.float32),
            pltpu.VMEM_SHARED((N, HD), jnp.float32),
        ] +
        [pltpu.SemaphoreType.DMA for _ in range(3 * SETS + 1)]
    ),
    compiler_params=pltpu.CompilerParams(use_tc_tiling_on_sc=False),
)


def _pad_edges(a):
    a2 = a.reshape(NS, TILE_EDGES)
    a2 = jnp.pad(a2, ((0, 0), (0, TILE_SPAN - TILE_EDGES)))
    return jnp.pad(a2.reshape(-1), (0, E_PAD - NS * TILE_SPAN))


@jax.jit
def kernel(user_table, group_table, adj_val, adj_row, adj_col,
           user_inputs, pos_groups, neg_groups):
    all0 = jnp.concatenate([user_table, group_table], axis=0)
    all0_cs = all0.reshape(N, NC, HD).transpose(1, 0, 2).reshape(NC * N, HD)
    adj_valp = _pad_edges(adj_val)
    adj_rowp2 = _pad_edges(adj_row).reshape(E_PAD // CHUNK, CHUNK)
    adj_colp = _pad_edges(adj_col)
    u_cs, p_cs, n_cs, u_ego, p_ego, n_ego, _ = _sc_call(
        all0_cs, user_table, group_table, adj_valp, adj_rowp2, adj_colp,
        user_inputs, pos_groups, neg_groups)
    user_embeds = jnp.concatenate([u_cs[0], u_cs[1]], axis=1)
    pos_embeds = jnp.concatenate([p_cs[0], p_cs[1]], axis=1)
    neg_embeds = jnp.concatenate([n_cs[0], n_cs[1]], axis=1)
    return (user_embeds, pos_embeds, neg_embeds, u_ego, p_ego, n_ego)


# speedup vs baseline: 11.3824x; 1.4611x over previous
import jax
import jax.numpy as jnp
from jax import lax
from jax.experimental import pallas as pl
from jax.experimental.pallas import tpu as pltpu
from jax.experimental.pallas import tpu_sc as plsc

NUM_USER = 30000
NUM_GROUP = 20000
N = NUM_USER + NUM_GROUP
E = 800000
D = 64
HD = D // 2
LAYERS = 3
B = 4096

NC = 2
NS = 16
CHUNK = 128
SETS = 4
TILE_EDGES = E // NS
NCHUNKS_TILE = 392
TILE_SPAN = NCHUNKS_TILE * CHUNK
E_PAD = NS * TILE_SPAN + 2 * CHUNK
RCHUNK = 100
NRCHUNKS = N // RCHUNK
EGO_CHUNK = 64
BCHUNK = 128


def _mul16(rows_ref, val_ref, n):
    for i in range(n // 16):
        e0 = i * 16
        vals = val_ref[pl.ds(e0, 16)]
        for j in range(16):
            v = vals[j]
            e = e0 + j
            rows_ref[e, pl.ds(0, 16)] = rows_ref[e, pl.ds(0, 16)] * v
            rows_ref[e, pl.ds(16, 16)] = rows_ref[e, pl.ds(16, 16)] * v


def _add_base(dst_ref, src_ref, base):
    for i in range(8):
        s = pl.ds(i * 16, 16)
        dst_ref[s] = src_ref[s] + base


def _sc_body(all0_cs, user_table, group_table, adj_valp, adj_rowp2, adj_colp,
             user_inputs, pos_groups, neg_groups,
             u_cs, p_cs, n_cs, u_ego, p_ego, n_ego, layers_hbm,
             col0, col1, col2, col3, row0, row1, row2, row3,
             val0, val1, val2, val3, rows0, rows1, rows2, rows3,
             tmpv, rowsum, bidx, boff, ebidx, egorows, acc_sh,
             semI0, semI1, semI2, semI3, semG0, semG1, semG2, semG3,
             semS0, semS1, semS2, semS3, gsem):
    c = lax.axis_index("c")
    s = lax.axis_index("s")

    cols = [col0, col1, col2, col3]
    rowsb = [row0, row1, row2, row3]
    vals = [val0, val1, val2, val3]
    rowsd = [rows0, rows1, rows2, rows3]
    semI = [semI0, semI1, semI2, semI3]
    semG = [semG0, semG1, semG2, semG3]
    semS = [semS0, semS1, semS2, semS3]

    nrchunks_tile = (NRCHUNKS - s + NS - 1) // NS
    base_e = s * TILE_SPAN
    base_ch = s * NCHUNKS_TILE

    def issue_idx(i, X):
        e0 = base_e + i * CHUNK
        pltpu.async_copy(adj_colp.at[pl.ds(e0, CHUNK)], cols[X], semI[X])
        pltpu.async_copy(adj_rowp2.at[pl.ds(base_ch + i, 1)], rowsb[X], semI[X])
        pltpu.async_copy(adj_valp.at[pl.ds(e0, CHUNK)], vals[X], semI[X])

    def drain_idx(X):
        pltpu.make_async_copy(adj_colp.at[pl.ds(0, CHUNK)], cols[X], semI[X]).wait()
        pltpu.make_async_copy(adj_rowp2.at[pl.ds(0, 1)], rowsb[X], semI[X]).wait()
        pltpu.make_async_copy(adj_valp.at[pl.ds(0, CHUNK)], vals[X], semI[X]).wait()

    def zero_rows(X):
        def zr(r, _):
            rowsd[X][r, pl.ds(0, 16)] = jnp.zeros((16,), jnp.float32)
            rowsd[X][r, pl.ds(16, 16)] = jnp.zeros((16,), jnp.float32)
            return 0
        lax.fori_loop(0, CHUNK, zr, 0)
        for ii in range(8):
            rowsb[X][0, pl.ds(ii * 16, 16)] = jnp.zeros((16,), jnp.int32)

    for l in range(1, LAYERS + 1):
        def zinit(i, _):
            tmpv[i, pl.ds(0, 16)] = jnp.zeros((16,), jnp.float32)
            tmpv[i, pl.ds(16, 16)] = jnp.zeros((16,), jnp.float32)
            return 0
        lax.fori_loop(0, RCHUNK, zinit, 0)

        def zbody(k, _):
            r0 = (s + k * NS) * RCHUNK
            pltpu.sync_copy(tmpv, acc_sh.at[pl.ds(r0, RCHUNK)])
            return 0
        lax.fori_loop(0, nrchunks_tile, zbody, 0)
        plsc.subcore_barrier()

        if l == 1:
            src_ref = all0_cs
            src_base = c * N
        else:
            src_ref = layers_hbm
            src_base = ((l - 2) * NC + c) * N

        def offs(X):
            for ii in range(8):
                sl = pl.ds(ii * 16, 16)
                cols[X][sl] = cols[X][sl] + src_base

        def issue_gather(X):
            pltpu.async_copy(src_ref.at[cols[X]], rowsd[X], semG[X])

        def drain_gather(X):
            pltpu.make_async_copy(src_ref.at[cols[X]], rowsd[X], semG[X]).wait()

        def issue_scatter(X):
            pltpu.async_copy(rowsd[X], acc_sh.at[rowsb[X].at[0]], semS[X],
                             add=True)

        def drain_scatter(X):
            pltpu.make_async_copy(rowsd[X], acc_sh.at[rowsb[X].at[0]],
                                  semS[X]).wait()

        for X in (2, 3):
            zero_rows(X)
            issue_scatter(X)
        issue_idx(0, 0)
        issue_idx(1, 1)
        drain_idx(0)
        offs(0)
        issue_gather(0)

        def round_body(k, _):
            for u in range(SETS):
                i = k * SETS + u
                X, Y, P = u, (u + 1) % SETS, (u + 2) % SETS
                drain_idx(Y)
                offs(Y)
                drain_scatter(P)
                issue_gather(Y)
                issue_idx(i + 2, P)
                drain_gather(X)
                _mul16(rowsd[X], vals[X], CHUNK)
                issue_scatter(X)
            return 0
        lax.fori_loop(0, NCHUNKS_TILE // SETS, round_body, 0)

        drain_scatter(2)
        drain_scatter(3)
        drain_gather(0)
        drain_idx(1)
        plsc.subcore_barrier()

        dst_base = ((l - 1) * NC + c) * N
        def fbody(k, _):
            r0 = (s + k * NS) * RCHUNK
            pltpu.sync_copy(acc_sh.at[pl.ds(r0, RCHUNK)], tmpv)
            pltpu.sync_copy(tmpv, layers_hbm.at[pl.ds(dst_base + r0, RCHUNK)])
            return 0
        lax.fori_loop(0, nrchunks_tile, fbody, 0)
        plsc.subcore_barrier()

    for idx_hbm, node_base, out_cs in (
        (user_inputs, 0, u_cs),
        (pos_groups, NUM_USER, p_cs),
        (neg_groups, NUM_USER, n_cs),
    ):
        for j in range(B // BCHUNK // NS):
            b0 = (s * (B // BCHUNK // NS) + j) * BCHUNK
            pltpu.sync_copy(idx_hbm.at[pl.ds(b0, BCHUNK)], bidx)
            for src_id in range(4):
                if src_id == 0:
                    base = node_base + c * N
                    src = all0_cs
                else:
                    base = node_base + ((src_id - 1) * NC + c) * N
                    src = layers_hbm
                _add_base(boff, bidx, base)
                pltpu.async_copy(src.at[boff], rows0, gsem).wait()

                def comb(r, _):
                    for g in range(2):
                        sl = pl.ds(g * 16, 16)
                        if src_id == 0:
                            rowsum[r, sl] = rows0[r, sl]
                        elif src_id == 3:
                            rowsum[r, sl] = (rowsum[r, sl] + rows0[r, sl]) * 0.25
                        else:
                            rowsum[r, sl] = rowsum[r, sl] + rows0[r, sl]
                    return 0
                lax.fori_loop(0, BCHUNK, comb, 0)
            pltpu.sync_copy(rowsum, out_cs.at[c, pl.ds(b0, BCHUNK)])

    w = s * NC + c
    for idx_hbm, table, out in (
        (user_inputs, user_table, u_ego),
        (pos_groups, group_table, p_ego),
        (neg_groups, group_table, n_ego),
    ):
        for h in range(BCHUNK // EGO_CHUNK):
            b0 = w * BCHUNK + h * EGO_CHUNK
            pltpu.sync_copy(idx_hbm.at[pl.ds(b0, EGO_CHUNK)], ebidx)
            pltpu.async_copy(table.at[ebidx], egorows, gsem).wait()
            pltpu.sync_copy(egorows, out.at[pl.ds(b0, EGO_CHUNK)])


_mesh = plsc.VectorSubcoreMesh(core_axis_name="c", subcore_axis_name="s",
                               num_cores=NC, num_subcores=NS)

_sc_call = pl.kernel(
    _sc_body,
    out_type=(
        jax.ShapeDtypeStruct((NC, B, HD), jnp.float32),
        jax.ShapeDtypeStruct((NC, B, HD), jnp.float32),
        jax.ShapeDtypeStruct((NC, B, HD), jnp.float32),
        jax.ShapeDtypeStruct((B, D), jnp.float32),
        jax.ShapeDtypeStruct((B, D), jnp.float32),
        jax.ShapeDtypeStruct((B, D), jnp.float32),
        jax.ShapeDtypeStruct((LAYERS * NC * N, HD), jnp.float32),
    ),
    mesh=_mesh,
    scratch_types=(
        [pltpu.VMEM((CHUNK,), jnp.int32) for _ in range(SETS)] +
        [pltpu.VMEM((1, CHUNK), jnp.int32) for _ in range(SETS)] +
        [pltpu.VMEM((CHUNK,), jnp.float32) for _ in range(SETS)] +
        [pltpu.VMEM((CHUNK, HD), jnp.float32) for _ in range(SETS)] +
        [
            pltpu.VMEM((RCHUNK, HD), jnp.float32),
            pltpu.VMEM((BCHUNK, HD), jnp.float32),
            pltpu.VMEM((BCHUNK,), jnp.int32),
            pltpu.VMEM((BCHUNK,), jnp.int32),
            pltpu.VMEM((EGO_CHUNK,), jnp.int32),
            pltpu.VMEM((EGO_CHUNK, D), jnp.float32),
            pltpu.VMEM_SHARED((N, HD), jnp.float32),
        ] +
        [pltpu.SemaphoreType.DMA for _ in range(3 * SETS + 1)]
    ),
    compiler_params=pltpu.CompilerParams(use_tc_tiling_on_sc=False),
)


def _pad_edges(a):
    a2 = a.reshape(NS, TILE_EDGES)
    a2 = jnp.pad(a2, ((0, 0), (0, TILE_SPAN - TILE_EDGES)))
    return jnp.pad(a2.reshape(-1), (0, E_PAD - NS * TILE_SPAN))


@jax.jit
def kernel(user_table, group_table, adj_val, adj_row, adj_col,
           user_inputs, pos_groups, neg_groups):
    all0 = jnp.concatenate([user_table, group_table], axis=0)
    all0_cs = all0.reshape(N, NC, HD).transpose(1, 0, 2).reshape(NC * N, HD)
    adj_valp = _pad_edges(adj_val)
    adj_rowp2 = _pad_edges(adj_row).reshape(E_PAD // CHUNK, CHUNK)
    adj_colp = _pad_edges(adj_col)
    u_cs, p_cs, n_cs, u_ego, p_ego, n_ego, _ = _sc_call(
        all0_cs, user_table, group_table, adj_valp, adj_rowp2, adj_colp,
        user_inputs, pos_groups, neg_groups)
    user_embeds = jnp.concatenate([u_cs[0], u_cs[1]], axis=1)
    pos_embeds = jnp.concatenate([p_cs[0], p_cs[1]], axis=1)
    neg_embeds = jnp.concatenate([n_cs[0], n_cs[1]], axis=1)
    return (user_embeds, pos_embeds, neg_embeds, u_ego, p_ego, n_ego)
